# baseline (speedup 1.0000x reference)
import functools

import jax
import jax.numpy as jnp
from jax import lax
from jax.experimental import pallas as pl
from jax.experimental.pallas import tpu as pltpu
from jax.experimental.pallas import tpu_sc as plsc

_B = 16384
_D = 64
_K = 20
_NC = 2
_NS = 16
_NW = _NC * _NS
_CHUNK = 128

_CEN_CH = _B // _NW // _CHUNK
_NEG_ROWS = _B * _K
_NEG_CH = _NEG_ROWS // _NW // _CHUNK


def _sc_gather_body(cen_idx, ctx_idx, neg_idx, w_cen, w_ctx,
                    out_cen, out_ctx, out_neg,
                    idx_cen_v, idx_ctx_v, idx_neg_v, buf, sem):
    wid = lax.axis_index("s") * _NC + lax.axis_index("c")
    pltpu.sync_copy(cen_idx.at[wid], idx_cen_v)
    pltpu.sync_copy(ctx_idx.at[wid], idx_ctx_v)
    pltpu.sync_copy(neg_idx.at[wid], idx_neg_v)

    def run(table, idx_v, nch, out_hbm):
        base = wid * nch * _CHUNK

        def body(j, carry):
            pltpu.async_copy(table.at[idx_v.at[j]], buf, sem).wait()
            pltpu.sync_copy(buf, out_hbm.at[pl.ds(base + j * _CHUNK, _CHUNK)])
            return carry

        lax.fori_loop(0, nch, body, 0)

    run(w_cen, idx_cen_v, _CEN_CH, out_cen)
    run(w_ctx, idx_ctx_v, _CEN_CH, out_ctx)
    run(w_ctx, idx_neg_v, _NEG_CH, out_neg)


_sc_gather = pl.kernel(
    _sc_gather_body,
    out_type=(
        jax.ShapeDtypeStruct((_B, _D), jnp.float32),
        jax.ShapeDtypeStruct((_B, _D), jnp.float32),
        jax.ShapeDtypeStruct((_NEG_ROWS, _D), jnp.float32),
    ),
    mesh=plsc.VectorSubcoreMesh(core_axis_name="c", subcore_axis_name="s",
                                num_cores=_NC, num_subcores=_NS),
    scratch_types=[
        pltpu.VMEM((_CEN_CH, _CHUNK), jnp.int32),
        pltpu.VMEM((_CEN_CH, _CHUNK), jnp.int32),
        pltpu.VMEM((_NEG_CH, _CHUNK), jnp.int32),
        pltpu.VMEM((_CHUNK, _D), jnp.float32),
        pltpu.SemaphoreType.DMA,
    ],
    compiler_params=pltpu.CompilerParams(use_tc_tiling_on_sc=False),
)


_BLK = 1024
_NBLK = _B // _BLK


def _log_sigmoid(x):
    return jnp.minimum(x, 0.0) - jnp.log1p(jnp.exp(-jnp.abs(x)))


def _tc_body(cen_ref, ctx_ref, neg_ref, out_ref, c_acc, s_acc):
    i = pl.program_id(0)

    @pl.when(i == 0)
    def _():
        c_acc[...] = jnp.zeros_like(c_acc)
        s_acc[0, 0] = 0.0

    cen = cen_ref[...]
    ctx = ctx_ref[...]
    neg = neg_ref[...]
    c_acc[...] += lax.dot_general(ctx, cen, (((0,), (0,)), ((), ())),
                                  preferred_element_type=jnp.float32)
    rest = jnp.sum(neg * cen[:, None, :], axis=-1)
    s_acc[0, 0] += jnp.sum(_log_sigmoid(-rest))

    @pl.when(i == _NBLK - 1)
    def _():
        out_ref[0, 0] = -(jnp.mean(_log_sigmoid(c_acc[...])) + s_acc[0, 0])


_tc_reduce = pl.pallas_call(
    _tc_body,
    grid=(_NBLK,),
    in_specs=[
        pl.BlockSpec((_BLK, _D), lambda i: (i, 0)),
        pl.BlockSpec((_BLK, _D), lambda i: (i, 0)),
        pl.BlockSpec((_BLK, _K, _D), lambda i: (i, 0, 0)),
    ],
    out_specs=pl.BlockSpec(memory_space=pltpu.SMEM),
    out_shape=jax.ShapeDtypeStruct((1, 1), jnp.float32),
    scratch_shapes=[
        pltpu.VMEM((_D, _D), jnp.float32),
        pltpu.SMEM((1, 1), jnp.float32),
    ],
)


def kernel(central, context, neg_samples, W_central, W_context):
    cen_r = central.astype(jnp.int32).reshape(_NW, _CEN_CH, _CHUNK)
    ctx_r = context.astype(jnp.int32).reshape(_NW, _CEN_CH, _CHUNK)
    neg_r = neg_samples.astype(jnp.int32).reshape(_NW, _NEG_CH, _CHUNK)
    emb_cen, emb_ctx, emb_neg = _sc_gather(cen_r, ctx_r, neg_r,
                                           W_central, W_context)
    loss = _tc_reduce(emb_cen, emb_ctx, emb_neg.reshape(_B, _K, _D))
    return loss.reshape(())
